# baseline (speedup 1.0000x reference)
import functools

import jax
import jax.numpy as jnp
from jax.experimental import pallas as pl
from jax.experimental.pallas import tpu as pltpu


def _se_kernel(x_ref, w1_ref, b1_ref, w2_ref, b2_ref, o_ref, *, inv_t, bb):
    for b in range(bb):
        xb = x_ref[b]
        s = jnp.sum(xb, axis=-1, keepdims=True)
        mean = s * jnp.float32(inv_t)
        h = jnp.dot(w1_ref[...], mean, preferred_element_type=jnp.float32)
        h = jnp.maximum(h + b1_ref[...], 0.0)
        g = jnp.dot(w2_ref[...], h, preferred_element_type=jnp.float32)
        gate = jax.nn.sigmoid(g + b2_ref[...])
        o_ref[b] = xb * gate


def kernel(x, w1, b1, w2, b2):
    B, C, T = x.shape
    H = w1.shape[0]
    BB = 4

    b1c = jnp.asarray(b1, jnp.float32).reshape(H, 1)
    b2c = jnp.asarray(b2, jnp.float32).reshape(C, 1)
    w1f = jnp.asarray(w1, jnp.float32)
    w2f = jnp.asarray(w2, jnp.float32)

    return pl.pallas_call(
        functools.partial(_se_kernel, inv_t=1.0 / T, bb=BB),
        out_shape=jax.ShapeDtypeStruct((B, C, T), x.dtype),
        grid=(B // BB,),
        in_specs=[
            pl.BlockSpec((BB, C, T), lambda b: (b, 0, 0)),
            pl.BlockSpec((H, C), lambda b: (0, 0)),
            pl.BlockSpec((H, 1), lambda b: (0, 0)),
            pl.BlockSpec((C, H), lambda b: (0, 0)),
            pl.BlockSpec((C, 1), lambda b: (0, 0)),
        ],
        out_specs=pl.BlockSpec((BB, C, T), lambda b: (b, 0, 0)),
        compiler_params=pltpu.CompilerParams(
            dimension_semantics=("parallel",),
            vmem_limit_bytes=64 * 1024 * 1024,
        ),
    )(x, w1f, b1c, w2f, b2c)


# speedup vs baseline: 1.0245x; 1.0245x over previous
import functools

import jax
import jax.numpy as jnp
from jax.experimental import pallas as pl
from jax.experimental.pallas import tpu as pltpu

_K = 4


def _se_kernel(x_hbm, w1t_ref, b1_ref, w2t_ref, b2_ref, o_hbm,
               xbuf, obuf, insem, outsem, *, inv_t, nb_per_core):
    core = pl.program_id(0)
    base = core * nb_per_core

    def dma_in(slot, i):
        return pltpu.make_async_copy(x_hbm.at[base + i], xbuf.at[slot],
                                     insem.at[slot])

    def dma_out(slot, i):
        return pltpu.make_async_copy(obuf.at[slot], o_hbm.at[base + i],
                                     outsem.at[slot])

    for k in range(min(_K, nb_per_core)):
        dma_in(k, k).start()

    def body(i, _):
        slot = jax.lax.rem(i, _K)
        dma_in(slot, i).wait()
        xv = xbuf.at[slot]
        xb = xv[...]
        mean = jnp.sum(xb.reshape(1, *xb.shape), axis=-1) * jnp.float32(inv_t)
        h = jnp.dot(mean, w1t_ref[...], preferred_element_type=jnp.float32)
        h = jnp.maximum(h + b1_ref[...], 0.0)
        s = jnp.dot(h, w2t_ref[...], preferred_element_type=jnp.float32)
        gate = jax.nn.sigmoid(s + b2_ref[...])

        @pl.when(i >= _K)
        def _():
            dma_out(slot, i - _K).wait()

        ov = obuf.at[slot]
        ov[...] = xb * gate.reshape(gate.shape[1], 1)
        dma_out(slot, i).start()

        @pl.when(i + _K < nb_per_core)
        def _():
            dma_in(slot, i + _K).start()

        return ()

    jax.lax.fori_loop(0, nb_per_core, body, ())

    for k in range(min(_K, nb_per_core)):
        dma_out(k, 0).wait()


def kernel(x, w1, b1, w2, b2):
    B, C, T = x.shape
    H = w1.shape[0]

    w1t = jnp.asarray(w1, jnp.float32).T
    w2t = jnp.asarray(w2, jnp.float32).T
    b1r = jnp.asarray(b1, jnp.float32).reshape(1, H)
    b2r = jnp.asarray(b2, jnp.float32).reshape(1, C)

    return pl.pallas_call(
        functools.partial(_se_kernel, inv_t=1.0 / T, nb_per_core=B // 2),
        out_shape=jax.ShapeDtypeStruct((B, C, T), x.dtype),
        grid=(2,),
        in_specs=[
            pl.BlockSpec(memory_space=pl.ANY),
            pl.BlockSpec((C, H), lambda b: (0, 0)),
            pl.BlockSpec((1, H), lambda b: (0, 0)),
            pl.BlockSpec((H, C), lambda b: (0, 0)),
            pl.BlockSpec((1, C), lambda b: (0, 0)),
        ],
        out_specs=pl.BlockSpec(memory_space=pl.ANY),
        scratch_shapes=[
            pltpu.VMEM((_K, C, T), jnp.float32),
            pltpu.VMEM((_K, C, T), jnp.float32),
            pltpu.SemaphoreType.DMA((_K,)),
            pltpu.SemaphoreType.DMA((_K,)),
        ],
        compiler_params=pltpu.CompilerParams(
            dimension_semantics=("parallel",),
            vmem_limit_bytes=64 * 1024 * 1024,
        ),
    )(x, w1t, b1r, w2t, b2r)
